# baseline (speedup 1.0000x reference)
import jax
import jax.numpy as jnp
from jax import lax
from jax.experimental import pallas as pl
from jax.experimental.pallas import tpu as pltpu
from jax.experimental.pallas import tpu_sc as plsc

_B, _F, _V = 16384, 26, 100000
_DE, _DH = 16, 32
_N = _B * _F
_NW = 32
_NPW = _N // _NW
_CH = 512
_NCH = _NPW // _CH
_L = 16


def _ser_body(fidx_hbm, easy_hbm, hard_hbm, easy_out, hard_out,
              fidx_v, rows_v, wide_v, oute_v, outh_v, sem):
    wid = lax.axis_index("s") * 2 + lax.axis_index("c")
    base = wid * _NPW

    pltpu.sync_copy(fidx_hbm.at[pl.ds(base, _NPW)], fidx_v)

    iota = lax.iota(jnp.int32, _L)

    def easy_chunk(c, carry):
        def prep(j, _):
            sl = pl.ds(j * _L, _L)
            rows_v[sl] = fidx_v[pl.ds(c * _CH + j * _L, _L)] >> 3
            return 0
        lax.fori_loop(0, _CH // _L, prep, 0)
        pltpu.async_copy(easy_hbm.at[rows_v], wide_v, sem).wait()

        def group(g, _):
            kvec = g * _L + iota
            fvec = fidx_v[pl.ds(c * _CH + g * _L, _L)]
            colb = (fvec & 7) * _DE
            dstb = kvec * _DE
            for l in range(_DE):
                vals = plsc.load_gather(wide_v, [kvec, colb + l])
                plsc.store_scatter(oute_v, [dstb + l], vals)
            return 0
        lax.fori_loop(0, _CH // _L, group, 0)
        pltpu.sync_copy(oute_v, easy_out.at[pl.ds((base + c * _CH) * _DE,
                                                  _CH * _DE)])
        return carry

    lax.fori_loop(0, _NCH, easy_chunk, 0)

    def hard_chunk(c, carry):
        def prep(j, _):
            sl = pl.ds(j * _L, _L)
            rows_v[sl] = fidx_v[pl.ds(c * _CH + j * _L, _L)] >> 2
            return 0
        lax.fori_loop(0, _CH // _L, prep, 0)
        pltpu.async_copy(hard_hbm.at[rows_v], wide_v, sem).wait()

        def group(g, _):
            kvec = g * _L + iota
            fvec = fidx_v[pl.ds(c * _CH + g * _L, _L)]
            colb = (fvec & 3) * _DH
            dstb = kvec * _DH
            for l in range(_DH):
                vals = plsc.load_gather(wide_v, [kvec, colb + l])
                plsc.store_scatter(outh_v, [dstb + l], vals)
            return 0
        lax.fori_loop(0, _CH // _L, group, 0)
        pltpu.sync_copy(outh_v, hard_out.at[pl.ds((base + c * _CH) * _DH,
                                                  _CH * _DH)])
        return carry

    lax.fori_loop(0, _NCH, hard_chunk, 0)


@jax.jit
def _ser(fidx, easy128, hard128):
    mesh = plsc.VectorSubcoreMesh(core_axis_name="c", subcore_axis_name="s")
    return pl.kernel(
        _ser_body,
        out_type=(
            jax.ShapeDtypeStruct((_N * _DE,), jnp.float32),
            jax.ShapeDtypeStruct((_N * _DH,), jnp.float32),
        ),
        mesh=mesh,
        scratch_types=[
            pltpu.VMEM((_NPW,), jnp.int32),
            pltpu.VMEM((_CH,), jnp.int32),
            pltpu.VMEM((_CH, 128), jnp.float32),
            pltpu.VMEM((_CH * _DE,), jnp.float32),
            pltpu.VMEM((_CH * _DH,), jnp.float32),
            pltpu.SemaphoreType.DMA,
        ],
        compiler_params=pltpu.CompilerParams(needs_layout_passes=False),
    )(fidx, easy128, hard128)


def kernel(X, easy_table, hard_table):
    fidx = (X + jnp.arange(_F, dtype=jnp.int32)[None, :] * _V).reshape(_N)
    easy128 = easy_table.reshape(_F * _V // 8, 128)
    hard128 = hard_table.reshape(_F * _V // 4, 128)
    easy_flat, hard_flat = _ser(fidx, easy128, hard128)
    return (easy_flat.reshape(_B, _F * _DE), hard_flat.reshape(_B, _F * _DH))


# speedup vs baseline: 1.3301x; 1.3301x over previous
import jax
import jax.numpy as jnp
from jax import lax
from jax.experimental import pallas as pl
from jax.experimental.pallas import tpu as pltpu
from jax.experimental.pallas import tpu_sc as plsc

_B, _F, _V = 16384, 26, 100000
_DE, _DH = 16, 32
_N = _B * _F
_NW = 32
_NPW = _N // _NW
_CH = 1024
_NCH = _NPW // _CH


def _ser_body(fidx_hbm, easy_hbm, hard_hbm, easy_out, hard_out,
              idx_v, easy_b, hard_b, sem_g):
    wid = lax.axis_index("s") * 2 + lax.axis_index("c")
    base = wid * _NPW

    pltpu.sync_copy(fidx_hbm.at[pl.ds(base, _NPW)], idx_v)

    def step(c, carry):
        sl = pl.ds(c * _CH, _CH)
        ce = pltpu.async_copy(easy_hbm.at[idx_v.at[sl]], easy_b, sem_g)
        ch = pltpu.async_copy(hard_hbm.at[idx_v.at[sl]], hard_b, sem_g)
        ce.wait()
        ch.wait()
        pltpu.sync_copy(easy_b, easy_out.at[pl.ds(base + c * _CH, _CH)])
        pltpu.sync_copy(hard_b, hard_out.at[pl.ds(base + c * _CH, _CH)])
        return carry

    lax.fori_loop(0, _NCH, step, 0)


@jax.jit
def _ser(fidx, easy_flat, hard_flat):
    mesh = plsc.VectorSubcoreMesh(core_axis_name="c", subcore_axis_name="s")
    return pl.kernel(
        _ser_body,
        out_type=(
            jax.ShapeDtypeStruct((_N, _DE), jnp.float32),
            jax.ShapeDtypeStruct((_N, _DH), jnp.float32),
        ),
        mesh=mesh,
        scratch_types=[
            pltpu.VMEM((_NPW,), jnp.int32),
            pltpu.VMEM((_CH, _DE), jnp.float32),
            pltpu.VMEM((_CH, _DH), jnp.float32),
            pltpu.SemaphoreType.DMA,
        ],
        compiler_params=pltpu.CompilerParams(use_tc_tiling_on_sc=False),
    )(fidx, easy_flat, hard_flat)


def kernel(X, easy_table, hard_table):
    fidx = (X + jnp.arange(_F, dtype=jnp.int32)[None, :] * _V).reshape(_N)
    easy_flat = easy_table.reshape(_F * _V, _DE)
    hard_flat = hard_table.reshape(_F * _V, _DH)
    easy_rows, hard_rows = _ser(fidx, easy_flat, hard_flat)
    return (easy_rows.reshape(_B, _F * _DE), hard_rows.reshape(_B, _F * _DH))


# speedup vs baseline: 1.3303x; 1.0001x over previous
import jax
import jax.numpy as jnp
from jax import lax
from jax.experimental import pallas as pl
from jax.experimental.pallas import tpu as pltpu
from jax.experimental.pallas import tpu_sc as plsc

_B, _F, _V = 16384, 26, 100000
_DE, _DH = 16, 32
_N = _B * _F
_NW = 32
_NPW = _N // _NW
_CH = 1024
_NCH = _NPW // _CH


def _ser_body(fidx_hbm, easy_hbm, hard_hbm, easy_out, hard_out,
              idx_v, easy_b, hard_b, sem_g):
    wid = lax.axis_index("s") * 2 + lax.axis_index("c")
    base = wid * _NPW

    pltpu.sync_copy(fidx_hbm.at[pl.ds(base, _NPW)], idx_v)

    def step(c, carry):
        sl = pl.ds(c * _CH, _CH)
        ce = pltpu.async_copy(easy_hbm.at[idx_v.at[sl]], easy_b, sem_g)
        ch = pltpu.async_copy(hard_hbm.at[idx_v.at[sl]], hard_b, sem_g)
        ce.wait()
        ch.wait()
        pltpu.sync_copy(easy_b, easy_out.at[pl.ds(base + c * _CH, _CH)])
        pltpu.sync_copy(hard_b, hard_out.at[pl.ds(base + c * _CH, _CH)])
        return carry

    lax.fori_loop(0, _NCH, step, 0)


@jax.jit
def _ser(fidx, easy_flat, hard_flat):
    mesh = plsc.VectorSubcoreMesh(core_axis_name="c", subcore_axis_name="s")
    return pl.kernel(
        _ser_body,
        out_type=(
            jax.ShapeDtypeStruct((_N, _DE), jnp.float32),
            jax.ShapeDtypeStruct((_N, _DH), jnp.float32),
        ),
        mesh=mesh,
        scratch_types=[
            pltpu.VMEM((_NPW,), jnp.int32),
            pltpu.VMEM((_CH, _DE), jnp.float32),
            pltpu.VMEM((_CH, _DH), jnp.float32),
            pltpu.SemaphoreType.DMA,
        ],
        compiler_params=pltpu.CompilerParams(use_tc_tiling_on_sc=False),
    )(fidx, easy_flat, hard_flat)


def kernel(X, easy_table, hard_table):
    fidx = (X + jnp.arange(_F, dtype=jnp.int32)[None, :] * _V).reshape(_N)
    one = (1 - 0 * X[0, 0]).astype(jnp.float32)
    easy_flat = easy_table.reshape(_F * _V, _DE) * one
    hard_flat = hard_table.reshape(_F * _V, _DH) * one
    easy_rows, hard_rows = _ser(fidx, easy_flat, hard_flat)
    return ((easy_rows * one).reshape(_B, _F * _DE),
            (hard_rows * one).reshape(_B, _F * _DH))
